# baseline (speedup 1.0000x reference)
import functools

import jax
import jax.numpy as jnp
from jax import lax
from jax.experimental import pallas as pl
from jax.experimental.pallas import tpu as pltpu
from jax.experimental.pallas import tpu_sc as plsc


def _emb_kernel(S, D, V, B, NC, NW):
    BLK = B // NW
    mesh = plsc.VectorSubcoreMesh(core_axis_name="c", subcore_axis_name="s")

    @functools.partial(
        pl.kernel,
        mesh=mesh,
        out_type=jax.ShapeDtypeStruct((S, D, B), jnp.float32),
        scratch_types=[
            pltpu.VMEM((V * D,), jnp.float32),
            pltpu.VMEM((BLK,), jnp.int32),
            pltpu.VMEM((BLK,), jnp.int32),
            pltpu.VMEM((D, BLK), jnp.float32),
            pltpu.VMEM((D, BLK), jnp.float32),
        ] + [pltpu.SemaphoreType.DMA for _ in range(10)],
        compiler_params=pltpu.CompilerParams(needs_layout_passes=False),
    )
    def k(wt_hbm, xt_hbm, out_hbm, wt_v, idx0, idx1, buf0, buf1, *sems):
        wid = lax.axis_index("s") * NC + lax.axis_index("c")
        col0 = wid * BLK
        idxs = (idx0, idx1)
        bufs = (buf0, buf1)
        isems = sems[:2]
        qsems = (sems[2:6], sems[6:10])

        pltpu.sync_copy(wt_hbm, wt_v)
        for h in range(2):
            pltpu.async_copy(xt_hbm.at[h, pl.ds(col0, BLK)], idxs[h],
                             isems[h])

        def body(i, carry):
            for h in range(2):
                s = 2 * i + h
                idx_v, buf = idxs[h], bufs[h]
                pltpu.make_async_copy(
                    xt_hbm.at[s, pl.ds(col0, BLK)], idx_v, isems[h]).wait()
                cvecs = [idx_v[pl.ds(g * 16, 16)] for g in range(BLK // 16)]

                @pl.when(i < (S // 2) - 1)
                def _():
                    pltpu.async_copy(
                        xt_hbm.at[s + 2, pl.ds(col0, BLK)], idx_v, isems[h])


                blocks = [(g, q * 16) for q in range(4)
                          for g in range(BLK // 16)]

                def qwrite(q):
                    pltpu.async_copy(
                        buf.at[pl.ds(q * 16, 16)],
                        out_hbm.at[s, pl.ds(q * 16, 16), pl.ds(col0, BLK)],
                        qsems[h][q])

                def qwait(q):
                    @pl.when(i > 0)
                    def _():
                        pltpu.make_async_copy(
                            buf.at[pl.ds(q * 16, 16)],
                            out_hbm.at[s, pl.ds(q * 16, 16),
                                       pl.ds(col0, BLK)],
                            qsems[h][q]).wait()

                prev = None
                for bi, (g, d0) in enumerate(blocks):
                    if bi % 8 == 0:
                        qwait(bi // 8)
                    cvec = cvecs[g]
                    cur = []
                    for u in range(16):
                        cur.append(
                            plsc.load_gather(wt_v, [cvec + (d0 + u) * V]))
                        if prev is not None:
                            pg, pd0, pvals = prev
                            buf[pd0 + u, pl.ds(pg * 16, 16)] = pvals[u]
                    prev = (g, d0, cur)
                    if bi % 8 == 0 and bi > 0:
                        qwrite(bi // 8 - 1)
                pg, pd0, pvals = prev
                for u in range(16):
                    buf[pd0 + u, pl.ds(pg * 16, 16)] = pvals[u]
                qwrite(3)
            return carry

        lax.fori_loop(0, S // 2, body, 0)
        for h in range(2):
            s = S - 2 + h
            for q in range(4):
                pltpu.make_async_copy(
                    bufs[h].at[pl.ds(q * 16, 16)],
                    out_hbm.at[s, pl.ds(q * 16, 16), pl.ds(col0, BLK)],
                    qsems[h][q]).wait()

    return k


def kernel(x, weights):
    Bdim, S = x.shape
    V, D = weights.shape
    info = plsc.get_sparse_core_info()
    NC, NS = info.num_cores, info.num_subcores
    NW = NC * NS
    wt_flat = weights.astype(jnp.float32).T.reshape(V * D)
    xt = x.astype(jnp.int32).T
    k = _emb_kernel(S, D, V, Bdim, NC, NW)
    out_t = k(wt_flat, xt)
    return jnp.transpose(out_t, (2, 0, 1))


# speedup vs baseline: 1.0031x; 1.0031x over previous
import functools

import jax
import jax.numpy as jnp
from jax import lax
from jax.experimental import pallas as pl
from jax.experimental.pallas import tpu as pltpu
from jax.experimental.pallas import tpu_sc as plsc


def _emb_kernel(S, D, V, B, NC, NW):
    BLK = B // NW
    mesh = plsc.VectorSubcoreMesh(core_axis_name="c", subcore_axis_name="s")

    @functools.partial(
        pl.kernel,
        mesh=mesh,
        out_type=jax.ShapeDtypeStruct((S, D, B), jnp.float32),
        scratch_types=[
            pltpu.VMEM((V * D,), jnp.float32),
            pltpu.VMEM((BLK,), jnp.int32),
            pltpu.VMEM((BLK,), jnp.int32),
            pltpu.VMEM((D, BLK), jnp.float32),
            pltpu.VMEM((D, BLK), jnp.float32),
            pltpu.SemaphoreType.DMA,
            pltpu.SemaphoreType.DMA,
            pltpu.SemaphoreType.DMA,
            pltpu.SemaphoreType.DMA,
        ],
        compiler_params=pltpu.CompilerParams(needs_layout_passes=False),
    )
    def k(wt_hbm, xt_hbm, out_hbm, wt_v, idx0, idx1, buf0, buf1,
          isem0, isem1, wsem0, wsem1):
        wid = lax.axis_index("s") * NC + lax.axis_index("c")
        col0 = wid * BLK
        idxs = (idx0, idx1)
        bufs = (buf0, buf1)
        isems = (isem0, isem1)
        wsems = (wsem0, wsem1)

        pltpu.sync_copy(wt_hbm, wt_v)
        for h in range(2):
            pltpu.async_copy(xt_hbm.at[h, pl.ds(col0, BLK)], idxs[h],
                             isems[h])

        def body(i, carry):
            for h in range(2):
                s = 2 * i + h
                idx_v, buf = idxs[h], bufs[h]
                pltpu.make_async_copy(
                    xt_hbm.at[s, pl.ds(col0, BLK)], idx_v, isems[h]).wait()
                cvecs = [idx_v[pl.ds(g * 16, 16)] for g in range(BLK // 16)]

                @pl.when(i < (S // 2) - 1)
                def _():
                    pltpu.async_copy(
                        xt_hbm.at[s + 2, pl.ds(col0, BLK)], idx_v, isems[h])

                @pl.when(i > 0)
                def _():
                    pltpu.make_async_copy(
                        buf, out_hbm.at[s, :, pl.ds(col0, BLK)],
                        wsems[h]).wait()

                blocks = [(g, q * 16) for q in range(4)
                          for g in range(BLK // 16)]
                prev = None
                for bi, (g, d0) in enumerate(blocks):
                    cvec = cvecs[g]
                    cur = []
                    for u in range(16):
                        cur.append(
                            plsc.load_gather(wt_v, [cvec + (d0 + u) * V]))
                        if prev is not None:
                            pg, pd0, pvals = prev
                            buf[pd0 + u, pl.ds(pg * 16, 16)] = pvals[u]
                    prev = (g, d0, cur)
                    if bi % 8 == 0 and bi > 0:
                        q = bi // 8 - 1
                        pltpu.async_copy(
                            buf.at[pl.ds(q * 16, 16)],
                            out_hbm.at[s, pl.ds(q * 16, 16),
                                       pl.ds(col0, BLK)],
                            wsems[h])
                pg, pd0, pvals = prev
                for u in range(16):
                    buf[pd0 + u, pl.ds(pg * 16, 16)] = pvals[u]
                pltpu.async_copy(
                    buf.at[pl.ds(48, 16)],
                    out_hbm.at[s, pl.ds(48, 16), pl.ds(col0, BLK)],
                    wsems[h])
            return carry

        lax.fori_loop(0, S // 2, body, 0)
        for h in range(2):
            s = S - 2 + h
            pltpu.make_async_copy(
                bufs[h], out_hbm.at[s, :, pl.ds(col0, BLK)], wsems[h]).wait()

    return k


def kernel(x, weights):
    Bdim, S = x.shape
    V, D = weights.shape
    info = plsc.get_sparse_core_info()
    NC, NS = info.num_cores, info.num_subcores
    NW = NC * NS
    wt_flat = weights.astype(jnp.float32).T.reshape(V * D)
    xt = x.astype(jnp.int32).T
    k = _emb_kernel(S, D, V, Bdim, NC, NW)
    out_t = k(wt_flat, xt)
    return jnp.transpose(out_t, (2, 0, 1))
